# Initial kernel scaffold; baseline (speedup 1.0000x reference)
import functools

import jax
import jax.numpy as jnp
from jax import lax
from jax.experimental import pallas as pl
from jax.experimental.pallas import tpu as pltpu
from jax.experimental.pallas import tpu_sc as plsc

EMB_DIM = 128
PART = EMB_DIM // 3
OFF_DIM = EMB_DIM - 2 * PART

B = 16384
NC, NS = 2, 16
NW = NC * NS
ROWS_PER_W = B // NW
CHUNK = 128
NCH = ROWS_PER_W // CHUNK


def _sc_embed(idx_s, idx_d, idx_o, scale_table, distortion_table, offset_table):
    mesh = plsc.VectorSubcoreMesh(core_axis_name="c", subcore_axis_name="s")

    @functools.partial(
        pl.kernel,
        out_type=jax.ShapeDtypeStruct((B, EMB_DIM), jnp.float32),
        mesh=mesh,
        scratch_types=[
            pltpu.VMEM((NCH, CHUNK), jnp.int32),
            pltpu.VMEM((NCH, CHUNK), jnp.int32),
            pltpu.VMEM((NCH, CHUNK), jnp.int32),
            pltpu.VMEM((CHUNK, PART), jnp.float32),
            pltpu.VMEM((CHUNK, PART), jnp.float32),
            pltpu.VMEM((CHUNK, OFF_DIM), jnp.float32),
            pltpu.VMEM((CHUNK, EMB_DIM), jnp.float32),
            pltpu.SemaphoreType.DMA,
        ],
    )
    def body(idx_s_hbm, idx_d_hbm, idx_o_hbm, scale_hbm, dist_hbm, off_hbm,
             out_hbm, idx_sv, idx_dv, idx_ov, rows_s, rows_d, rows_o, comb, sem):
        wid = lax.axis_index("s") * NC + lax.axis_index("c")
        base = wid * ROWS_PER_W
        pltpu.sync_copy(idx_s_hbm.at[wid], idx_sv)
        pltpu.sync_copy(idx_d_hbm.at[wid], idx_dv)
        pltpu.sync_copy(idx_o_hbm.at[wid], idx_ov)
        for j in range(NCH):
            cs = pltpu.async_copy(scale_hbm.at[idx_sv.at[j]], rows_s, sem)
            cd = pltpu.async_copy(dist_hbm.at[idx_dv.at[j]], rows_d, sem)
            co = pltpu.async_copy(off_hbm.at[idx_ov.at[j]], rows_o, sem)
            cs.wait()
            cd.wait()
            co.wait()
            pltpu.sync_copy(rows_s, comb.at[:, pl.ds(0, PART)])
            pltpu.sync_copy(rows_d, comb.at[:, pl.ds(PART, PART)])
            pltpu.sync_copy(rows_o, comb.at[:, pl.ds(2 * PART, OFF_DIM)])
            pltpu.sync_copy(comb, out_hbm.at[pl.ds(base + j * CHUNK, CHUNK), :])

    return body(idx_s, idx_d, idx_o, scale_table, distortion_table, offset_table)


@jax.jit
def kernel(scale_conditions, distortion_conditions, offset_conditions,
           scale_table, distortion_table, offset_table):
    idx_s = scale_conditions.astype(jnp.int32).reshape(NW, NCH, CHUNK)
    idx_d = distortion_conditions.astype(jnp.int32).reshape(NW, NCH, CHUNK)
    idx_o = offset_conditions.astype(jnp.int32).reshape(NW, NCH, CHUNK)
    return _sc_embed(idx_s, idx_d, idx_o, scale_table, distortion_table,
                     offset_table)


# speedup vs baseline: 6.3310x; 6.3310x over previous
import functools

import jax
import jax.numpy as jnp
from jax import lax
from jax.experimental import pallas as pl
from jax.experimental.pallas import tpu as pltpu
from jax.experimental.pallas import tpu_sc as plsc

EMB_DIM = 128
PART = EMB_DIM // 3
OFF_DIM = EMB_DIM - 2 * PART

B = 16384
NC, NS, LANES = 2, 16, 16
NW = NC * NS
ROWS_PER_W = B // NW
CHUNK = 128
NCH = ROWS_PER_W // CHUNK
N_SD = 2 * 7
N_FUSED = N_SD * 200


def _sc_embed(idx_s, idx_d, idx_o, fused_table):
    mesh = plsc.VectorSubcoreMesh(core_axis_name="c", subcore_axis_name="s")

    @functools.partial(
        pl.kernel,
        out_type=jax.ShapeDtypeStruct((B, EMB_DIM), jnp.float32),
        mesh=mesh,
        scratch_types=[
            pltpu.VMEM((NCH, CHUNK), jnp.int32),
            pltpu.VMEM((NCH, CHUNK), jnp.int32),
            pltpu.VMEM((NCH, CHUNK), jnp.int32),
            pltpu.VMEM((NCH, CHUNK), jnp.int32),
            pltpu.VMEM((CHUNK, EMB_DIM), jnp.float32),
            pltpu.SemaphoreType.DMA,
        ],
    )
    def body(idx_s_hbm, idx_d_hbm, idx_o_hbm, ft_hbm, out_hbm,
             idx_sv, idx_dv, idx_ov, idx_fv, comb, sem):
        wid = lax.axis_index("s") * NC + lax.axis_index("c")
        base = wid * ROWS_PER_W
        pltpu.sync_copy(idx_s_hbm.at[wid], idx_sv)
        pltpu.sync_copy(idx_d_hbm.at[wid], idx_dv)
        pltpu.sync_copy(idx_o_hbm.at[wid], idx_ov)
        for j in range(NCH):
            for k in range(CHUNK // LANES):
                sl = pl.ds(k * LANES, LANES)
                sv = idx_sv[j, sl]
                dv = idx_dv[j, sl]
                ov = idx_ov[j, sl]
                idx_fv[j, sl] = (sv * (7 * 200) + dv * 200) + ov
        for j in range(NCH):
            pltpu.async_copy(ft_hbm.at[idx_fv.at[j]], comb, sem).wait()
            pltpu.sync_copy(comb, out_hbm.at[pl.ds(base + j * CHUNK, CHUNK), :])

    return body(idx_s, idx_d, idx_o, fused_table)


@jax.jit
def kernel(scale_conditions, distortion_conditions, offset_conditions,
           scale_table, distortion_table, offset_table):
    idx_s = scale_conditions.astype(jnp.int32).reshape(NW, NCH, CHUNK)
    idx_d = distortion_conditions.astype(jnp.int32).reshape(NW, NCH, CHUNK)
    idx_o = offset_conditions.astype(jnp.int32).reshape(NW, NCH, CHUNK)
    s_b = jnp.broadcast_to(scale_table[:, None, None, :], (2, 7, 200, PART))
    d_b = jnp.broadcast_to(distortion_table[None, :, None, :], (2, 7, 200, PART))
    o_b = jnp.broadcast_to(offset_table[None, None, :, :], (2, 7, 200, OFF_DIM))
    fused_table = jnp.concatenate([s_b, d_b, o_b], axis=-1).reshape(
        N_FUSED, EMB_DIM)
    return _sc_embed(idx_s, idx_d, idx_o, fused_table)


# speedup vs baseline: 7.0551x; 1.1144x over previous
import functools

import jax
import jax.numpy as jnp
from jax import lax
from jax.experimental import pallas as pl
from jax.experimental.pallas import tpu as pltpu
from jax.experimental.pallas import tpu_sc as plsc

EMB_DIM = 128
PART = EMB_DIM // 3
OFF_DIM = EMB_DIM - 2 * PART

B = 16384
NC, NS, LANES = 2, 16, 16
NW = NC * NS
ROWS_PER_W = B // NW
CHUNK = 128
NCH = ROWS_PER_W // CHUNK
N_SD = 2 * 7
N_FUSED = N_SD * 200


def _sc_embed(idx_s, idx_d, idx_o, fused_table):
    mesh = plsc.VectorSubcoreMesh(core_axis_name="c", subcore_axis_name="s")

    @functools.partial(
        pl.kernel,
        out_type=jax.ShapeDtypeStruct((B, EMB_DIM), jnp.float32),
        mesh=mesh,
        scratch_types=[
            pltpu.VMEM((NCH, CHUNK), jnp.int32),
            pltpu.VMEM((NCH, CHUNK), jnp.int32),
            pltpu.VMEM((NCH, CHUNK), jnp.int32),
            pltpu.VMEM((NCH, CHUNK), jnp.int32),
            pltpu.VMEM((NCH, CHUNK, EMB_DIM), jnp.float32),
            pltpu.SemaphoreType.DMA,
            pltpu.SemaphoreType.DMA,
            pltpu.SemaphoreType.DMA,
        ],
    )
    def body(idx_s_hbm, idx_d_hbm, idx_o_hbm, ft_hbm, out_hbm,
             idx_sv, idx_dv, idx_ov, idx_fv, comb, sem_i, sem_g, sem_w):
        wid = lax.axis_index("s") * NC + lax.axis_index("c")
        base = wid * ROWS_PER_W
        ci = [pltpu.async_copy(idx_s_hbm.at[wid], idx_sv, sem_i),
              pltpu.async_copy(idx_d_hbm.at[wid], idx_dv, sem_i),
              pltpu.async_copy(idx_o_hbm.at[wid], idx_ov, sem_i)]
        for c in ci:
            c.wait()
        for j in range(NCH):
            for k in range(CHUNK // LANES):
                sl = pl.ds(k * LANES, LANES)
                sv = idx_sv[j, sl]
                dv = idx_dv[j, sl]
                ov = idx_ov[j, sl]
                idx_fv[j, sl] = (sv * (7 * 200) + dv * 200) + ov
        gs = [pltpu.async_copy(ft_hbm.at[idx_fv.at[j]], comb.at[j], sem_g)
              for j in range(NCH)]
        ws = []
        for j in range(NCH):
            gs[j].wait()
            ws.append(pltpu.async_copy(
                comb.at[j], out_hbm.at[pl.ds(base + j * CHUNK, CHUNK), :],
                sem_w))
        for w in ws:
            w.wait()

    return body(idx_s, idx_d, idx_o, fused_table)


@jax.jit
def kernel(scale_conditions, distortion_conditions, offset_conditions,
           scale_table, distortion_table, offset_table):
    idx_s = scale_conditions.astype(jnp.int32).reshape(NW, NCH, CHUNK)
    idx_d = distortion_conditions.astype(jnp.int32).reshape(NW, NCH, CHUNK)
    idx_o = offset_conditions.astype(jnp.int32).reshape(NW, NCH, CHUNK)
    s_b = jnp.broadcast_to(scale_table[:, None, None, :], (2, 7, 200, PART))
    d_b = jnp.broadcast_to(distortion_table[None, :, None, :], (2, 7, 200, PART))
    o_b = jnp.broadcast_to(offset_table[None, None, :, :], (2, 7, 200, OFF_DIM))
    fused_table = jnp.concatenate([s_b, d_b, o_b], axis=-1).reshape(
        N_FUSED, EMB_DIM)
    return _sc_embed(idx_s, idx_d, idx_o, fused_table)


# speedup vs baseline: 7.0769x; 1.0031x over previous
import functools

import jax
import jax.numpy as jnp
from jax import lax
from jax.experimental import pallas as pl
from jax.experimental.pallas import tpu as pltpu
from jax.experimental.pallas import tpu_sc as plsc

EMB_DIM = 128
PART = EMB_DIM // 3
OFF_DIM = EMB_DIM - 2 * PART

B = 16384
NC, NS, LANES = 2, 16, 16
NW = NC * NS
ROWS_PER_W = B // NW
CHUNK = 128
NCH = ROWS_PER_W // CHUNK
N_FUSED = 2 * 7 * 200


def _sc_embed(idx_s, idx_d, idx_o, fused_table):
    mesh = plsc.VectorSubcoreMesh(core_axis_name="c", subcore_axis_name="s")

    @functools.partial(
        pl.kernel,
        out_type=jax.ShapeDtypeStruct((B, EMB_DIM), jnp.float32),
        mesh=mesh,
        scratch_types=[
            pltpu.VMEM((ROWS_PER_W,), jnp.int32),
            pltpu.VMEM((ROWS_PER_W,), jnp.int32),
            pltpu.VMEM((ROWS_PER_W,), jnp.int32),
            pltpu.VMEM((ROWS_PER_W,), jnp.int32),
            pltpu.VMEM((NCH, CHUNK, EMB_DIM), jnp.float32),
            pltpu.SemaphoreType.DMA,
            pltpu.SemaphoreType.DMA,
            pltpu.SemaphoreType.DMA,
        ],
    )
    def body(idx_s_hbm, idx_d_hbm, idx_o_hbm, ft_hbm, out_hbm,
             idx_sv, idx_dv, idx_ov, idx_fv, comb, sem_i, sem_g, sem_w):
        wid = lax.axis_index("s") * NC + lax.axis_index("c")
        base = wid * ROWS_PER_W
        ci = [pltpu.async_copy(idx_s_hbm.at[pl.ds(base, ROWS_PER_W)], idx_sv,
                               sem_i),
              pltpu.async_copy(idx_d_hbm.at[pl.ds(base, ROWS_PER_W)], idx_dv,
                               sem_i),
              pltpu.async_copy(idx_o_hbm.at[pl.ds(base, ROWS_PER_W)], idx_ov,
                               sem_i)]
        for c in ci:
            c.wait()
        for k in range(ROWS_PER_W // LANES):
            sl = pl.ds(k * LANES, LANES)
            idx_fv[sl] = (idx_sv[sl] * (7 * 200) + idx_dv[sl] * 200) + idx_ov[sl]
        gs = [pltpu.async_copy(ft_hbm.at[idx_fv.at[pl.ds(j * CHUNK, CHUNK)]],
                               comb.at[j], sem_g)
              for j in range(NCH)]
        ws = []
        for j in range(NCH):
            gs[j].wait()
            ws.append(pltpu.async_copy(
                comb.at[j], out_hbm.at[pl.ds(base + j * CHUNK, CHUNK), :],
                sem_w))
        for w in ws:
            w.wait()

    return body(idx_s, idx_d, idx_o, fused_table)


@jax.jit
def kernel(scale_conditions, distortion_conditions, offset_conditions,
           scale_table, distortion_table, offset_table):
    idx_s = scale_conditions.astype(jnp.int32)
    idx_d = distortion_conditions.astype(jnp.int32)
    idx_o = offset_conditions.astype(jnp.int32)
    s_b = jnp.broadcast_to(scale_table[:, None, None, :], (2, 7, 200, PART))
    d_b = jnp.broadcast_to(distortion_table[None, :, None, :], (2, 7, 200, PART))
    o_b = jnp.broadcast_to(offset_table[None, None, :, :], (2, 7, 200, OFF_DIM))
    fused_table = jnp.concatenate([s_b, d_b, o_b], axis=-1).reshape(
        N_FUSED, EMB_DIM)
    return _sc_embed(idx_s, idx_d, idx_o, fused_table)
